# baseline (speedup 1.0000x reference)
import functools

import jax
import jax.numpy as jnp
import numpy as np
from jax.experimental import pallas as pl

N, S, D, C, TOPK = 64, 2048, 128, 1024, 64
NB = 8


def _logits_topk_body(q_ref, K_ref, w_ref, wk_ref, idx_ref):
    q = q_ref[...]
    K = K_ref[...].reshape(NB * S, D)
    P = jax.lax.dot_general(q, K, (((1,), (1,)), ((), ())),
                            preferred_element_type=jnp.float32)
    P = P.reshape(NB, NB, S)
    nid = jax.lax.broadcasted_iota(jnp.int32, (NB, NB, 1), 0)
    mid = jax.lax.broadcasted_iota(jnp.int32, (NB, NB, 1), 1)
    L = jnp.sum(jnp.where(nid == mid, P, 0.0), axis=1)
    L = L * np.float32(1.0 / np.sqrt(D))

    iota_s = jax.lax.broadcasted_iota(jnp.int32, (NB, S), 1)
    iota_k = jax.lax.broadcasted_iota(jnp.int32, (NB, TOPK), 1)
    neg_inf = jnp.float32(-jnp.inf)

    work = L
    vmax = jnp.max(work, axis=-1, keepdims=True)
    acc = jnp.zeros((NB, S), jnp.float32)
    denom = jnp.zeros((NB, 1), jnp.float32)
    wk_acc = jnp.zeros((NB, TOPK), jnp.float32)
    idx_acc = jnp.zeros((NB, TOPK), jnp.int32)

    for t in range(TOPK):
        m = jnp.max(work, axis=-1, keepdims=True)
        eq = work == m
        pos = jnp.min(jnp.where(eq, iota_s, S), axis=-1, keepdims=True)
        onehot = iota_s == pos
        e = jnp.exp(m - vmax)
        acc = acc + jnp.where(onehot, e, 0.0)
        denom = denom + e
        sel = iota_k == t
        wk_acc = jnp.where(sel, e, wk_acc)
        idx_acc = jnp.where(sel, pos, idx_acc)
        work = jnp.where(onehot, neg_inf, work)

    w_ref[...] = acc / denom
    wk_ref[...] = wk_acc / denom
    row = pl.program_id(0) * NB + jax.lax.broadcasted_iota(jnp.int32, (NB, TOPK), 0)
    idx_ref[...] = idx_acc + row * S


def _summary_dense_body(w_ref, V_ref, s_ref):
    w = w_ref[...]
    V = V_ref[...]
    s_ref[...] = jnp.sum(V * w[:, :, None], axis=1)


def _readout_body(s_ref, Wc_ref, bc_ref, Wr_ref, br_ref, cls_ref, rec_ref):
    s = s_ref[...]
    cls_ref[...] = jax.lax.dot_general(
        s, Wc_ref[...], (((1,), (1,)), ((), ())),
        preferred_element_type=jnp.float32) + bc_ref[...]
    rec_ref[...] = jax.lax.dot_general(
        s, Wr_ref[...], (((1,), (1,)), ((), ())),
        preferred_element_type=jnp.float32) + br_ref[...]


@jax.jit
def kernel(q, K, V, z, y, W_c, b_c, W_r, b_r):
    del z, y
    grid = (N // NB,)
    weights, wk, idxg = pl.pallas_call(
        _logits_topk_body,
        grid=grid,
        in_specs=[
            pl.BlockSpec((NB, D), lambda i: (i, 0)),
            pl.BlockSpec((NB, S, D), lambda i: (i, 0, 0)),
        ],
        out_specs=[
            pl.BlockSpec((NB, S), lambda i: (i, 0)),
            pl.BlockSpec((NB, TOPK), lambda i: (i, 0)),
            pl.BlockSpec((NB, TOPK), lambda i: (i, 0)),
        ],
        out_shape=[
            jax.ShapeDtypeStruct((N, S), jnp.float32),
            jax.ShapeDtypeStruct((N, TOPK), jnp.float32),
            jax.ShapeDtypeStruct((N, TOPK), jnp.int32),
        ],
    )(q, K)

    summary = pl.pallas_call(
        _summary_dense_body,
        grid=grid,
        in_specs=[
            pl.BlockSpec((NB, S), lambda i: (i, 0)),
            pl.BlockSpec((NB, S, D), lambda i: (i, 0, 0)),
        ],
        out_specs=pl.BlockSpec((NB, D), lambda i: (i, 0)),
        out_shape=jax.ShapeDtypeStruct((N, D), jnp.float32),
    )(weights, V)
    del wk, idxg

    cls_out, rec_out = pl.pallas_call(
        _readout_body,
        in_specs=[
            pl.BlockSpec((N, D), lambda: (0, 0)),
            pl.BlockSpec((C, D), lambda: (0, 0)),
            pl.BlockSpec((1, C), lambda: (0, 0)),
            pl.BlockSpec((D, D), lambda: (0, 0)),
            pl.BlockSpec((1, D), lambda: (0, 0)),
        ],
        out_specs=[
            pl.BlockSpec((N, C), lambda: (0, 0)),
            pl.BlockSpec((N, D), lambda: (0, 0)),
        ],
        out_shape=[
            jax.ShapeDtypeStruct((N, C), jnp.float32),
            jax.ShapeDtypeStruct((N, D), jnp.float32),
        ],
    )(summary, W_c, b_c.reshape(1, C), W_r, b_r.reshape(1, D))

    return (cls_out, rec_out, weights)


# speedup vs baseline: 1.7301x; 1.7301x over previous
import functools

import jax
import jax.numpy as jnp
import numpy as np
from jax import lax
from jax.experimental import pallas as pl
from jax.experimental.pallas import tpu as pltpu
from jax.experimental.pallas import tpu_sc as plsc

N, S, D, C, TOPK = 64, 2048, 128, 1024, 64
NB = 8
NWORKERS = 32
ROWS_PER_W = N // NWORKERS
INT_MIN = np.int32(-2147483648)


def _logits_select_body(q_ref, K_ref, w_ref):
    q = q_ref[...]
    K = K_ref[...].reshape(NB * S, D)
    P = lax.dot_general(q, K, (((1,), (1,)), ((), ())),
                        preferred_element_type=jnp.float32)
    P = P.reshape(NB, NB, S)
    nid = lax.broadcasted_iota(jnp.int32, (NB, NB, 1), 0)
    mid = lax.broadcasted_iota(jnp.int32, (NB, NB, 1), 1)
    L = jnp.sum(jnp.where(nid == mid, P, 0.0), axis=1)
    L = L * np.float32(1.0 / np.sqrt(D))

    b = lax.bitcast_convert_type(L, jnp.int32)
    key = b ^ jnp.where(b < 0, jnp.int32(0x7FFFFFFF), jnp.int32(0))

    def count_ge(t):
        return jnp.sum(jnp.where(key >= t, jnp.int32(1), jnp.int32(0)),
                       axis=-1, keepdims=True)

    T = jnp.where(count_ge(jnp.zeros((NB, 1), jnp.int32)) >= TOPK,
                  jnp.int32(0), INT_MIN)
    for bit in range(30, -1, -1):
        Tc = T | jnp.int32(1 << bit)
        T = jnp.where(count_ge(Tc) >= TOPK, Tc, T)

    gt = key > T
    eq = key == T
    cnt_gt = jnp.sum(jnp.where(gt, jnp.int32(1), jnp.int32(0)),
                     axis=-1, keepdims=True)
    r = TOPK - cnt_gt
    eqi = jnp.where(eq, jnp.int32(1), jnp.int32(0))
    pos = lax.broadcasted_iota(jnp.int32, (NB, S), 1)
    csum = eqi
    sft = 1
    while sft < S:
        csum = csum + jnp.where(pos >= sft, pltpu.roll(csum, sft, 1),
                                jnp.int32(0))
        sft *= 2
    rank = csum - eqi
    sel = gt | (eq & (rank < r))

    rowmax = jnp.max(L, axis=-1, keepdims=True)
    ex = jnp.where(sel, jnp.exp(L - rowmax), 0.0)
    denom = jnp.sum(ex, axis=-1, keepdims=True)
    w_ref[...] = ex / denom


def _sc_body(W_hbm, V_hbm, G_hbm, wk_hbm, wrow_v, idx_v, wkv_v, rows_v, sem):
    wid = lax.axis_index("s") * 2 + lax.axis_index("c")
    lanes = lax.iota(jnp.int32, 16)
    for rr in range(ROWS_PER_W):
        n = wid * ROWS_PER_W + rr
        pltpu.sync_copy(W_hbm.at[n], wrow_v)
        for j in range(TOPK // 16):
            idx_v[pl.ds(16 * j, 16)] = jnp.zeros((16,), jnp.int32)
            wkv_v[pl.ds(16 * j, 16)] = jnp.zeros((16,), jnp.float32)

        def chunk(c, off):
            w16 = wrow_v[pl.ds(c * 16, 16)]
            m = w16 > 0.0
            ones = jnp.where(m, jnp.int32(1), jnp.int32(0))
            csum = plsc.cumsum(ones)
            cnt = jnp.sum(ones)
            dst = jnp.where(m, off + csum - 1, jnp.int32(0))
            spos = n * S + c * 16 + lanes
            plsc.store_scatter(idx_v, [dst], spos, mask=m)
            plsc.store_scatter(wkv_v, [dst], w16, mask=m)
            return off + cnt

        lax.fori_loop(0, S // 16, chunk, jnp.int32(0))

        pltpu.async_copy(V_hbm.at[idx_v], rows_v, sem).wait()
        pltpu.sync_copy(rows_v, G_hbm.at[pl.ds(n * TOPK, TOPK)])
        pltpu.sync_copy(wkv_v, wk_hbm.at[n])


def _make_sc_compact_gather():
    return functools.partial(
        pl.kernel,
        mesh=plsc.VectorSubcoreMesh(core_axis_name="c", subcore_axis_name="s"),
        compiler_params=pltpu.CompilerParams(needs_layout_passes=False),
        out_type=[
            jax.ShapeDtypeStruct((N * TOPK, D), jnp.float32),
            jax.ShapeDtypeStruct((N, TOPK), jnp.float32),
        ],
        scratch_types=[
            pltpu.VMEM((S,), jnp.float32),
            pltpu.VMEM((TOPK,), jnp.int32),
            pltpu.VMEM((TOPK,), jnp.float32),
            pltpu.VMEM((TOPK, D), jnp.float32),
            pltpu.SemaphoreType.DMA,
        ],
    )(_sc_body)


def _readout_body(wk_ref, G_ref, Wc_ref, bc_ref, Wr_ref, br_ref,
                  cls_ref, rec_ref):
    G = G_ref[...].reshape(N, TOPK, D)
    wk = wk_ref[...]
    s = jnp.sum(G * wk[:, :, None], axis=1)
    cls_ref[...] = lax.dot_general(
        s, Wc_ref[...], (((1,), (1,)), ((), ())),
        preferred_element_type=jnp.float32) + bc_ref[...]
    rec_ref[...] = lax.dot_general(
        s, Wr_ref[...], (((1,), (1,)), ((), ())),
        preferred_element_type=jnp.float32) + br_ref[...]


@jax.jit
def kernel(q, K, V, z, y, W_c, b_c, W_r, b_r):
    del z, y
    weights = pl.pallas_call(
        _logits_select_body,
        grid=(N // NB,),
        in_specs=[
            pl.BlockSpec((NB, D), lambda i: (i, 0)),
            pl.BlockSpec((NB, S, D), lambda i: (i, 0, 0)),
        ],
        out_specs=pl.BlockSpec((NB, S), lambda i: (i, 0)),
        out_shape=jax.ShapeDtypeStruct((N, S), jnp.float32),
    )(q, K)

    G, wk = _make_sc_compact_gather()(weights, V.reshape(N * S, D))

    cls_out, rec_out = pl.pallas_call(
        _readout_body,
        in_specs=[
            pl.BlockSpec((N, TOPK), lambda: (0, 0)),
            pl.BlockSpec((N * TOPK, D), lambda: (0, 0)),
            pl.BlockSpec((C, D), lambda: (0, 0)),
            pl.BlockSpec((1, C), lambda: (0, 0)),
            pl.BlockSpec((D, D), lambda: (0, 0)),
            pl.BlockSpec((1, D), lambda: (0, 0)),
        ],
        out_specs=[
            pl.BlockSpec((N, C), lambda: (0, 0)),
            pl.BlockSpec((N, D), lambda: (0, 0)),
        ],
        out_shape=[
            jax.ShapeDtypeStruct((N, C), jnp.float32),
            jax.ShapeDtypeStruct((N, D), jnp.float32),
        ],
    )(wk, G, W_c, b_c.reshape(1, C), W_r, b_r.reshape(1, D))

    return (cls_out, rec_out, weights)


# speedup vs baseline: 2.1461x; 1.2404x over previous
import functools

import jax
import jax.numpy as jnp
import numpy as np
from jax import lax
from jax.experimental import pallas as pl
from jax.experimental.pallas import tpu as pltpu
from jax.experimental.pallas import tpu_sc as plsc

N, S, D, C, TOPK = 64, 2048, 128, 1024, 64
NB = 8
NWORKERS = 32
ROWS_PER_W = N // NWORKERS
INT_MIN = np.int32(-2147483648)


def _logits_select_body(q_ref, K_ref, w_ref):
    rows = [
        lax.dot_general(q_ref[i:i + 1, :], K_ref[i],
                        (((1,), (1,)), ((), ())),
                        preferred_element_type=jnp.float32)
        for i in range(NB)
    ]
    L = jnp.concatenate(rows, axis=0) * np.float32(1.0 / np.sqrt(D))

    b = lax.bitcast_convert_type(L, jnp.int32)
    key = b ^ jnp.where(b < 0, jnp.int32(0x7FFFFFFF), jnp.int32(0))

    def count_ge(t):
        return jnp.sum(jnp.where(key >= t, jnp.int32(1), jnp.int32(0)),
                       axis=-1, keepdims=True)

    T = jnp.where(count_ge(jnp.zeros((NB, 1), jnp.int32)) >= TOPK,
                  jnp.int32(0), INT_MIN)
    for bit in range(30, -1, -1):
        Tc = T | jnp.int32(1 << bit)
        T = jnp.where(count_ge(Tc) >= TOPK, Tc, T)

    gt = key > T
    eq = key == T
    cnt_gt = jnp.sum(jnp.where(gt, jnp.int32(1), jnp.int32(0)),
                     axis=-1, keepdims=True)
    r = TOPK - cnt_gt
    eqi = jnp.where(eq, jnp.int32(1), jnp.int32(0))
    NCH = S // 128
    e3 = eqi.reshape(NB, NCH, 128)
    lane = lax.broadcasted_iota(jnp.int32, (NB, NCH, 128), 2)
    c = e3
    for sft in (1, 2, 4, 8, 16, 32, 64):
        c = c + jnp.where(lane >= sft, pltpu.roll(c, sft, 2), jnp.int32(0))
    tot = c[:, :, 127]
    ch = lax.broadcasted_iota(jnp.int32, (NB, NCH), 1)
    t2 = tot
    for sft in (1, 2, 4, 8):
        t2 = t2 + jnp.where(ch >= sft, pltpu.roll(t2, sft, 1), jnp.int32(0))
    excl = t2 - tot
    rank = (c - e3 + excl[:, :, None]).reshape(NB, S)
    sel = gt | (eq & (rank < r))

    rowmax = jnp.max(L, axis=-1, keepdims=True)
    ex = jnp.where(sel, jnp.exp(L - rowmax), 0.0)
    denom = jnp.sum(ex, axis=-1, keepdims=True)
    w_ref[...] = ex / denom


def _sc_body(W_hbm, V_hbm, G_hbm, wk_hbm, wrow_v, idx_v, wkv_v, rows_v, sem):
    wid = lax.axis_index("s") * 2 + lax.axis_index("c")
    lanes = lax.iota(jnp.int32, 16)
    for rr in range(ROWS_PER_W):
        n = wid * ROWS_PER_W + rr
        pltpu.sync_copy(W_hbm.at[n], wrow_v)
        for j in range(TOPK // 16):
            idx_v[pl.ds(16 * j, 16)] = jnp.zeros((16,), jnp.int32)
            wkv_v[pl.ds(16 * j, 16)] = jnp.zeros((16,), jnp.float32)

        def chunk(c, off):
            w16 = wrow_v[pl.ds(c * 16, 16)]
            m = w16 > 0.0
            ones = jnp.where(m, jnp.int32(1), jnp.int32(0))
            csum = plsc.cumsum(ones)
            cnt = jnp.sum(ones)
            dst = jnp.where(m, off + csum - 1, jnp.int32(0))
            spos = n * S + c * 16 + lanes
            plsc.store_scatter(idx_v, [dst], spos, mask=m)
            plsc.store_scatter(wkv_v, [dst], w16, mask=m)
            return off + cnt

        lax.fori_loop(0, S // 16, chunk, jnp.int32(0))

        pltpu.async_copy(V_hbm.at[idx_v], rows_v, sem).wait()
        pltpu.sync_copy(rows_v, G_hbm.at[pl.ds(n * TOPK, TOPK)])
        pltpu.sync_copy(wkv_v, wk_hbm.at[n])


def _make_sc_compact_gather():
    return functools.partial(
        pl.kernel,
        mesh=plsc.VectorSubcoreMesh(core_axis_name="c", subcore_axis_name="s"),
        compiler_params=pltpu.CompilerParams(needs_layout_passes=False),
        out_type=[
            jax.ShapeDtypeStruct((N * TOPK, D), jnp.float32),
            jax.ShapeDtypeStruct((N, TOPK), jnp.float32),
        ],
        scratch_types=[
            pltpu.VMEM((S,), jnp.float32),
            pltpu.VMEM((TOPK,), jnp.int32),
            pltpu.VMEM((TOPK,), jnp.float32),
            pltpu.VMEM((TOPK, D), jnp.float32),
            pltpu.SemaphoreType.DMA,
        ],
    )(_sc_body)


def _readout_body(wk_ref, G_ref, Wc_ref, bc_ref, Wr_ref, br_ref,
                  cls_ref, rec_ref):
    G = G_ref[...].reshape(N, TOPK, D)
    wk = wk_ref[...]
    s = jnp.sum(G * wk[:, :, None], axis=1)
    cls_ref[...] = lax.dot_general(
        s, Wc_ref[...], (((1,), (1,)), ((), ())),
        preferred_element_type=jnp.float32) + bc_ref[...]
    rec_ref[...] = lax.dot_general(
        s, Wr_ref[...], (((1,), (1,)), ((), ())),
        preferred_element_type=jnp.float32) + br_ref[...]


@jax.jit
def kernel(q, K, V, z, y, W_c, b_c, W_r, b_r):
    del z, y
    weights = pl.pallas_call(
        _logits_select_body,
        grid=(N // NB,),
        in_specs=[
            pl.BlockSpec((NB, D), lambda i: (i, 0)),
            pl.BlockSpec((NB, S, D), lambda i: (i, 0, 0)),
        ],
        out_specs=pl.BlockSpec((NB, S), lambda i: (i, 0)),
        out_shape=jax.ShapeDtypeStruct((N, S), jnp.float32),
    )(q, K)

    G, wk = _make_sc_compact_gather()(weights, V.reshape(N * S, D))

    cls_out, rec_out = pl.pallas_call(
        _readout_body,
        in_specs=[
            pl.BlockSpec((N, TOPK), lambda: (0, 0)),
            pl.BlockSpec((N * TOPK, D), lambda: (0, 0)),
            pl.BlockSpec((C, D), lambda: (0, 0)),
            pl.BlockSpec((1, C), lambda: (0, 0)),
            pl.BlockSpec((D, D), lambda: (0, 0)),
            pl.BlockSpec((1, D), lambda: (0, 0)),
        ],
        out_specs=[
            pl.BlockSpec((N, C), lambda: (0, 0)),
            pl.BlockSpec((N, D), lambda: (0, 0)),
        ],
        out_shape=[
            jax.ShapeDtypeStruct((N, C), jnp.float32),
            jax.ShapeDtypeStruct((N, D), jnp.float32),
        ],
    )(wk, G, W_c, b_c.reshape(1, C), W_r, b_r.reshape(1, D))

    return (cls_out, rec_out, weights)
